# Initial kernel scaffold; baseline (speedup 1.0000x reference)
import jax
import jax.numpy as jnp
from jax.experimental import pallas as pl


def kernel(x, W):
    raise NotImplementedError("write your pallas kernel here")


# speedup vs baseline: 1.2441x; 1.2441x over previous
import functools

import jax
import jax.numpy as jnp
from jax import lax
from jax.experimental import pallas as pl
from jax.experimental.pallas import tpu as pltpu
from jax.experimental.pallas import tpu_sc as plsc

D_MODEL = 512
BATCH = 4096 * 50

_info = plsc.get_sparse_core_info()
_NC, _NS = _info.num_cores, _info.num_subcores
_NW = _NC * _NS
_B_PER_W = BATCH // _NW
_CHUNK = 80
_N_CHUNKS = _B_PER_W // _CHUNK


def _build():
    mesh = plsc.VectorSubcoreMesh(core_axis_name="c", subcore_axis_name="s")

    @functools.partial(
        pl.kernel,
        out_type=jax.ShapeDtypeStruct((BATCH, D_MODEL), jnp.float32),
        mesh=mesh,
        scratch_types=[
            pltpu.VMEM((_B_PER_W,), jnp.int32),
            pltpu.VMEM((_CHUNK, D_MODEL), jnp.float32),
            pltpu.SemaphoreType.DMA,
        ],
    )
    def emb(idx_hbm, table_hbm, out_hbm, idx_v, buf, sem):
        wid = lax.axis_index("s") * _NC + lax.axis_index("c")
        base = wid * _B_PER_W
        pltpu.sync_copy(idx_hbm.at[pl.ds(base, _B_PER_W)], idx_v)

        def body(g, carry):
            row0 = g * _CHUNK
            pltpu.async_copy(
                table_hbm.at[idx_v.at[pl.ds(row0, _CHUNK)]], buf, sem
            ).wait()
            pltpu.sync_copy(buf, out_hbm.at[pl.ds(base + row0, _CHUNK)])
            return carry

        lax.fori_loop(0, _N_CHUNKS, body, 0)

    return emb


_emb = _build()


def kernel(x, W):
    flat = x.reshape(-1).astype(jnp.int32)
    out = _emb(flat, W)
    return out.reshape(x.shape + (D_MODEL,))


# speedup vs baseline: 1.2968x; 1.0423x over previous
import functools

import jax
import jax.numpy as jnp
from jax import lax
from jax.experimental import pallas as pl
from jax.experimental.pallas import tpu as pltpu
from jax.experimental.pallas import tpu_sc as plsc

D_MODEL = 512
BATCH = 4096 * 50

_info = plsc.get_sparse_core_info()
_NC, _NS = _info.num_cores, _info.num_subcores
_NW = _NC * _NS
_B_PER_W = BATCH // _NW
_CHUNK = 80
_N_CHUNKS = _B_PER_W // _CHUNK


def _build():
    mesh = plsc.VectorSubcoreMesh(core_axis_name="c", subcore_axis_name="s")

    @functools.partial(
        pl.kernel,
        out_type=jax.ShapeDtypeStruct((BATCH, D_MODEL), jnp.float32),
        mesh=mesh,
        scratch_types=[
            pltpu.VMEM((_B_PER_W,), jnp.int32),
            pltpu.VMEM((_CHUNK, D_MODEL), jnp.float32),
            pltpu.VMEM((_CHUNK, D_MODEL), jnp.float32),
            pltpu.SemaphoreType.DMA,
            pltpu.SemaphoreType.DMA,
            pltpu.SemaphoreType.DMA,
            pltpu.SemaphoreType.DMA,
        ],
    )
    def emb(idx_hbm, table_hbm, out_hbm, idx_v, buf0, buf1,
            gsem0, gsem1, wsem0, wsem1):
        wid = lax.axis_index("s") * _NC + lax.axis_index("c")
        base = wid * _B_PER_W
        pltpu.sync_copy(idx_hbm.at[pl.ds(base, _B_PER_W)], idx_v)

        def g_start(g, buf, sem):
            pltpu.make_async_copy(
                table_hbm.at[idx_v.at[pl.ds(g * _CHUNK, _CHUNK)]], buf, sem
            ).start()

        def g_wait(buf, sem):
            pltpu.make_async_copy(
                table_hbm.at[idx_v.at[pl.ds(0, _CHUNK)]], buf, sem
            ).wait()

        def w_start(g, buf, sem):
            pltpu.make_async_copy(
                buf, out_hbm.at[pl.ds(base + g * _CHUNK, _CHUNK)], sem
            ).start()

        def w_wait(buf, sem):
            pltpu.make_async_copy(
                buf, out_hbm.at[pl.ds(base, _CHUNK)], sem
            ).wait()

        n_pairs = _N_CHUNKS // 2
        g_start(0, buf0, gsem0)
        g_start(1, buf1, gsem1)

        def body(i, carry):
            g0 = 2 * i
            g_wait(buf0, gsem0)
            w_start(g0, buf0, wsem0)
            g_wait(buf1, gsem1)
            w_start(g0 + 1, buf1, wsem1)

            @pl.when(i < n_pairs - 1)
            def _next():
                w_wait(buf0, wsem0)
                g_start(g0 + 2, buf0, gsem0)
                w_wait(buf1, wsem1)
                g_start(g0 + 3, buf1, gsem1)

            return carry

        lax.fori_loop(0, n_pairs, body, 0)
        w_wait(buf0, wsem0)
        w_wait(buf1, wsem1)

    return emb


_emb = _build()


def kernel(x, W):
    flat = x.reshape(-1).astype(jnp.int32)
    out = _emb(flat, W)
    return out.reshape(x.shape + (D_MODEL,))


# speedup vs baseline: 1.9031x; 1.4676x over previous
import functools

import jax
import jax.numpy as jnp
from jax import lax
from jax.experimental import pallas as pl
from jax.experimental.pallas import tpu as pltpu
from jax.experimental.pallas import tpu_sc as plsc

D_MODEL = 512
N_ROWS = 4096
N_COLS = 50

_info = plsc.get_sparse_core_info()
_NC, _NS = _info.num_cores, _info.num_subcores
_NW = _NC * _NS
_ROWS_PER_W = N_ROWS // _NW


def _build():
    mesh = plsc.VectorSubcoreMesh(core_axis_name="c", subcore_axis_name="s")

    @functools.partial(
        pl.kernel,
        out_type=jax.ShapeDtypeStruct((N_ROWS, N_COLS, D_MODEL), jnp.float32),
        mesh=mesh,
        scratch_types=[
            pltpu.VMEM((_ROWS_PER_W, N_COLS), jnp.int32),
            pltpu.VMEM((N_COLS, D_MODEL), jnp.float32),
            pltpu.VMEM((N_COLS, D_MODEL), jnp.float32),
            pltpu.SemaphoreType.DMA,
            pltpu.SemaphoreType.DMA,
            pltpu.SemaphoreType.DMA,
            pltpu.SemaphoreType.DMA,
        ],
    )
    def emb(idx_hbm, table_hbm, out_hbm, idx_v, buf0, buf1,
            gsem0, gsem1, wsem0, wsem1):
        wid = lax.axis_index("s") * _NC + lax.axis_index("c")
        base = wid * _ROWS_PER_W
        pltpu.sync_copy(idx_hbm.at[pl.ds(base, _ROWS_PER_W)], idx_v)

        def g_start(g, buf, sem):
            pltpu.make_async_copy(table_hbm.at[idx_v.at[g]], buf, sem).start()

        def g_wait(buf, sem):
            pltpu.make_async_copy(table_hbm.at[idx_v.at[0]], buf, sem).wait()

        def w_start(g, buf, sem):
            pltpu.make_async_copy(buf, out_hbm.at[base + g], sem).start()

        def w_wait(buf, sem):
            pltpu.make_async_copy(buf, out_hbm.at[base], sem).wait()

        n_pairs = _ROWS_PER_W // 2
        g_start(0, buf0, gsem0)
        g_start(1, buf1, gsem1)

        def body(i, carry):
            g0 = 2 * i
            g_wait(buf0, gsem0)
            w_start(g0, buf0, wsem0)
            g_wait(buf1, gsem1)
            w_start(g0 + 1, buf1, wsem1)

            @pl.when(i < n_pairs - 1)
            def _next():
                w_wait(buf0, wsem0)
                g_start(g0 + 2, buf0, gsem0)
                w_wait(buf1, wsem1)
                g_start(g0 + 3, buf1, gsem1)

            return carry

        lax.fori_loop(0, n_pairs, body, 0)
        w_wait(buf0, wsem0)
        w_wait(buf1, wsem1)

    return emb


_emb = _build()


def kernel(x, W):
    return _emb(x.astype(jnp.int32), W)


# speedup vs baseline: 1.9197x; 1.0087x over previous
import functools

import jax
import jax.numpy as jnp
from jax import lax
from jax.experimental import pallas as pl
from jax.experimental.pallas import tpu as pltpu
from jax.experimental.pallas import tpu_sc as plsc

D_MODEL = 512
N_ROWS = 4096
N_COLS = 50
PAD_COLS = 64

_info = plsc.get_sparse_core_info()
_NC, _NS = _info.num_cores, _info.num_subcores
_NW = _NC * _NS
_ROWS_PER_W = N_ROWS // _NW
_NBUF = 4


def _build():
    mesh = plsc.VectorSubcoreMesh(core_axis_name="c", subcore_axis_name="s")

    @functools.partial(
        pl.kernel,
        out_type=jax.ShapeDtypeStruct((N_ROWS, N_COLS, D_MODEL), jnp.float32),
        mesh=mesh,
        scratch_types=[
            pltpu.VMEM((_ROWS_PER_W * PAD_COLS,), jnp.int32),
        ]
        + [pltpu.VMEM((N_COLS, D_MODEL), jnp.float32)] * _NBUF
        + [pltpu.SemaphoreType.DMA] * (2 * _NBUF),
    )
    def emb(idx_hbm, table_hbm, out_hbm, idx_v, *bufs_and_sems):
        bufs = bufs_and_sems[:_NBUF]
        gsems = bufs_and_sems[_NBUF:2 * _NBUF]
        wsems = bufs_and_sems[2 * _NBUF:]
        wid = lax.axis_index("s") * _NC + lax.axis_index("c")
        base = wid * _ROWS_PER_W
        pltpu.sync_copy(
            idx_hbm.at[pl.ds(base * PAD_COLS, _ROWS_PER_W * PAD_COLS)], idx_v)

        def g_start(g, b):
            pltpu.make_async_copy(
                table_hbm.at[idx_v.at[pl.ds(g * PAD_COLS, N_COLS)]],
                bufs[b], gsems[b]).start()

        def g_wait(b):
            pltpu.make_async_copy(
                table_hbm.at[idx_v.at[pl.ds(0, N_COLS)]],
                bufs[b], gsems[b]).wait()

        def w_start(g, b):
            pltpu.make_async_copy(bufs[b], out_hbm.at[base + g], wsems[b]).start()

        def w_wait(b):
            pltpu.make_async_copy(bufs[b], out_hbm.at[base], wsems[b]).wait()

        n_steps = _ROWS_PER_W // _NBUF
        for b in range(_NBUF):
            g_start(b, b)

        def body(i, carry):
            g0 = i * _NBUF
            for b in range(_NBUF):
                g_wait(b)
                w_start(g0 + b, b)

            @pl.when(i < n_steps - 1)
            def _next():
                for b in range(_NBUF):
                    w_wait(b)
                    g_start(g0 + _NBUF + b, b)

            return carry

        lax.fori_loop(0, n_steps, body, 0)
        for b in range(_NBUF):
            w_wait(b)

    return emb


_emb = _build()


def kernel(x, W):
    xp = jnp.pad(x.astype(jnp.int32), ((0, 0), (0, PAD_COLS - N_COLS)))
    return _emb(xp.reshape(-1), W)


# speedup vs baseline: 4.1556x; 2.1647x over previous
import functools

import jax
import jax.numpy as jnp
from jax import lax
from jax.experimental import pallas as pl
from jax.experimental.pallas import tpu as pltpu
from jax.experimental.pallas import tpu_sc as plsc

D_MODEL = 512
N_ROWS = 4096
N_COLS = 50

_info = plsc.get_sparse_core_info()
_NC, _NS = _info.num_cores, _info.num_subcores
_NW = _NC * _NS
_IBLK = N_ROWS // _NW
_CHUNK = 32
_NBUF = 5
_CPP = _IBLK // _CHUNK
_N_ITEMS = N_COLS * _CPP


def _build():
    mesh = plsc.VectorSubcoreMesh(core_axis_name="c", subcore_axis_name="s")

    @functools.partial(
        pl.kernel,
        out_type=jax.ShapeDtypeStruct((N_COLS, N_ROWS, D_MODEL), jnp.float32),
        mesh=mesh,
        scratch_types=[
            pltpu.VMEM((N_COLS, _IBLK), jnp.int32),
            pltpu.SemaphoreType.DMA,
        ]
        + [pltpu.VMEM((_CHUNK, D_MODEL), jnp.float32)] * _NBUF
        + [pltpu.SemaphoreType.DMA] * (2 * _NBUF),
    )
    def emb(idx_hbm, table_hbm, out_hbm, idx_v, isem, *bufs_and_sems):
        bufs = bufs_and_sems[:_NBUF]
        gsems = bufs_and_sems[_NBUF:2 * _NBUF]
        wsems = bufs_and_sems[2 * _NBUF:]
        wid = lax.axis_index("s") * _NC + lax.axis_index("c")
        ibase = wid * _IBLK

        pltpu.make_async_copy(
            idx_hbm.at[:, pl.ds(ibase, _IBLK)], idx_v, isem).start()
        pltpu.make_async_copy(
            idx_hbm.at[:, pl.ds(0, _IBLK)], idx_v, isem).wait()

        def g_start(t, b):
            pltpu.make_async_copy(
                table_hbm.at[idx_v.at[t // _CPP, pl.ds((t % _CPP) * _CHUNK, _CHUNK)]],
                bufs[b], gsems[b]).start()

        def g_wait(b):
            pltpu.make_async_copy(
                table_hbm.at[idx_v.at[0, pl.ds(0, _CHUNK)]],
                bufs[b], gsems[b]).wait()

        def w_start(t, b):
            j = t // _CPP
            c = t % _CPP
            pltpu.make_async_copy(
                bufs[b], out_hbm.at[j, pl.ds(ibase + c * _CHUNK, _CHUNK)],
                wsems[b]).start()

        def w_wait(b):
            pltpu.make_async_copy(
                bufs[b], out_hbm.at[0, pl.ds(0, _CHUNK)], wsems[b]).wait()

        n_steps = _N_ITEMS // _NBUF
        ahead = 3
        drain_lag = _NBUF - ahead
        for k in range(ahead):
            g_start(k, k)

        def body(i, carry):
            t0 = i * _NBUF
            for b in range(_NBUF):
                t = t0 + b
                g_wait(b)
                w_start(t, b)
                bn = (b + ahead) % _NBUF

                @pl.when(t + ahead < _N_ITEMS)
                def _arm():
                    @pl.when(t >= drain_lag)
                    def _drain():
                        w_wait(bn)

                    g_start(t + ahead, bn)

            return carry

        lax.fori_loop(0, n_steps, body, 0)
        for b in range(_NBUF):
            w_wait(b)

    return emb


_emb = _build()


def kernel(x, W):
    xt = jnp.transpose(x.astype(jnp.int32))
    return jnp.transpose(_emb(xt, W), (1, 0, 2))


# speedup vs baseline: 4.1623x; 1.0016x over previous
import functools

import jax
import jax.numpy as jnp
from jax import lax
from jax.experimental import pallas as pl
from jax.experimental.pallas import tpu as pltpu
from jax.experimental.pallas import tpu_sc as plsc

D_MODEL = 512
N_ROWS = 4096
N_COLS = 50

_info = plsc.get_sparse_core_info()
_NC, _NS = _info.num_cores, _info.num_subcores
_NW = _NC * _NS
_IBLK = N_ROWS // _NW
_CHUNK = 32
_NBUF = 6
_CPP = _IBLK // _CHUNK
_N_ITEMS = N_COLS * _CPP


def _build():
    mesh = plsc.VectorSubcoreMesh(core_axis_name="c", subcore_axis_name="s")

    @functools.partial(
        pl.kernel,
        out_type=jax.ShapeDtypeStruct((N_COLS, N_ROWS, D_MODEL), jnp.float32),
        mesh=mesh,
        scratch_types=[
            pltpu.VMEM((N_COLS, _IBLK), jnp.int32),
            pltpu.SemaphoreType.DMA,
        ]
        + [pltpu.VMEM((_CHUNK, D_MODEL), jnp.float32)] * _NBUF
        + [pltpu.SemaphoreType.DMA] * (2 * _NBUF),
    )
    def emb(idx_hbm, table_hbm, out_hbm, idx_v, isem, *bufs_and_sems):
        bufs = bufs_and_sems[:_NBUF]
        gsems = bufs_and_sems[_NBUF:2 * _NBUF]
        wsems = bufs_and_sems[2 * _NBUF:]
        wid = lax.axis_index("s") * _NC + lax.axis_index("c")
        ibase = wid * _IBLK

        pltpu.make_async_copy(
            idx_hbm.at[:, pl.ds(ibase, _IBLK)], idx_v, isem).start()
        pltpu.make_async_copy(
            idx_hbm.at[:, pl.ds(0, _IBLK)], idx_v, isem).wait()

        def g_start(t, b):
            pltpu.make_async_copy(
                table_hbm.at[idx_v.at[t // _CPP, pl.ds((t % _CPP) * _CHUNK, _CHUNK)]],
                bufs[b], gsems[b]).start()

        def g_wait(b):
            pltpu.make_async_copy(
                table_hbm.at[idx_v.at[0, pl.ds(0, _CHUNK)]],
                bufs[b], gsems[b]).wait()

        def w_start(t, b):
            j = t // _CPP
            c = t % _CPP
            pltpu.make_async_copy(
                bufs[b], out_hbm.at[j, pl.ds(ibase + c * _CHUNK, _CHUNK)],
                wsems[b]).start()

        def w_wait(b):
            pltpu.make_async_copy(
                bufs[b], out_hbm.at[0, pl.ds(0, _CHUNK)], wsems[b]).wait()

        n_steps = _N_ITEMS // _NBUF
        ahead = 4
        drain_lag = _NBUF - ahead
        for k in range(ahead):
            g_start(k, k)

        def body(i, carry):
            t0 = i * _NBUF
            for b in range(_NBUF):
                t = t0 + b
                g_wait(b)
                w_start(t, b)
                bn = (b + ahead) % _NBUF

                @pl.when(t + ahead < _N_ITEMS)
                def _arm():
                    @pl.when(t >= drain_lag)
                    def _drain():
                        w_wait(bn)

                    g_start(t + ahead, bn)

            return carry

        lax.fori_loop(0, n_steps, body, 0)
        for t in range(n_steps * _NBUF, _N_ITEMS):
            g_wait(t % _NBUF)
            w_start(t, t % _NBUF)
        for b in range(_NBUF):
            w_wait(b)

    return emb


_emb = _build()


def kernel(x, W):
    xt = jnp.transpose(x.astype(jnp.int32))
    return jnp.transpose(_emb(xt, W), (1, 0, 2))
